# Initial kernel scaffold; baseline (speedup 1.0000x reference)
import jax
import jax.numpy as jnp
from jax.experimental import pallas as pl


def kernel(x, edge_index, w_in1, b1, w_out1, w_in2, b2, w_out2):
    raise NotImplementedError("write your pallas kernel here")


# speedup vs baseline: 5.7442x; 5.7442x over previous
import jax
import jax.numpy as jnp
from jax import lax
from jax.experimental import pallas as pl
from jax.experimental.pallas import tpu as pltpu
from jax.experimental.pallas import tpu_sc as plsc

N_FUNC = 40000
N_IN = 5000
N_OUT = 5000
NUM_NODES = N_FUNC + N_IN + N_OUT
CH = 4
E_FF = 640000
E_IF = 80000
E_FO = 80000
B = 16
BH = 8
ROW = BH * CH
CK = 128
C_FF = 5008
C_IF = 640
C_FO = 640
EC = 80
NCH = N_FUNC // EC
OUTR = 1280


def _body(x2_r, idxFF_r, wFF1_r, wFF2_r, idxIF_r, wIF1_r, wIF2_r,
          idxFO_r, wFO1_r, wFO2_r, biasx_r,
          out_r, hid_r,
          zb, idxb, idxb4, wb1, wb2, gb, gb2, cb, b8, ab, bb, idv,
          acc, outacc, sem1, sem2):
    c = lax.axis_index("c")
    s = lax.axis_index("s")
    iota = lax.iota(jnp.int32, 16)
    zero16 = jnp.zeros((16,), jnp.float32)

    def full16(v):
        return jnp.full((16,), v, jnp.int32)

    def zrow(r, _):
        zb[r, pl.ds(0, 16)] = zero16
        zb[r, pl.ds(16, 16)] = zero16
        return 0
    lax.fori_loop(0, EC, zrow, 0)

    nch_t = jnp.where(s < NCH % 16, NCH // 16 + 1, NCH // 16)

    def zacc(u, _):
        pltpu.sync_copy(zb, acc.at[pl.ds((u * 16 + s) * EC, EC)])
        return 0
    lax.fori_loop(0, nch_t, zacc, 0)
    pltpu.sync_copy(zb, outacc.at[pl.ds(s * (OUTR // 16), OUTR // 16)])
    plsc.subcore_barrier()

    def if_pass(w_ref):
        def body(i, _):
            ck = i * 16 + s
            pltpu.sync_copy(idxIF_r.at[c, ck], idxb)
            pltpu.sync_copy(w_ref.at[pl.ds(ck * CK, CK)], wb1)
            pltpu.async_copy(x2_r.at[idxb.at[0]], b8, sem1).wait()
            for g in range(8):
                ridx = iota + g * 16
                wk = [plsc.load_gather(wb1, [ridx, full16(k)])
                      for k in range(CH)]
                for b in range(BH):
                    xg = plsc.load_gather(b8, [ridx, full16(b)])
                    for k in range(CH):
                        plsc.store_scatter(cb, [ridx, full16(b * CH + k)],
                                           xg * wk[k])
            pltpu.sync_copy(cb, acc.at[idxb.at[1]], add=True)
            return 0
        lax.fori_loop(0, C_IF // 16, body, 0)

    def elu_pass(layer, zero_after):
        def chunk(u, _):
            row0 = (u * 16 + s) * EC
            pltpu.sync_copy(acc.at[pl.ds(row0, EC)], ab)
            pltpu.sync_copy(biasx_r.at[layer, pl.ds(row0, EC)], bb)

            def erow(r, _):
                for h in (0, 16):
                    v = ab[r, pl.ds(h, 16)] + bb[r, pl.ds(h, 16)]
                    ab[r, pl.ds(h, 16)] = jnp.where(
                        v > 0.0, v, jnp.exp(jnp.minimum(v, 0.0)) - 1.0)
                return 0
            lax.fori_loop(0, EC, erow, 0)
            pltpu.sync_copy(
                ab, hid_r.at[pl.ds((layer * 2 + c) * N_FUNC + row0, EC)])
            if zero_after:
                pltpu.sync_copy(zb, acc.at[pl.ds(row0, EC)])
            return 0
        lax.fori_loop(0, nch_t, chunk, 0)

    def ff_pass():
        def body(i, _):
            ck = i * 16 + s
            pltpu.sync_copy(idxFF_r.at[c, ck], idxb)
            pltpu.sync_copy(wFF1_r.at[pl.ds(ck * CK, CK)], wb1)
            pltpu.sync_copy(wFF2_r.at[pl.ds(ck * CK, CK)], wb2)
            pltpu.async_copy(hid_r.at[idxb.at[0]], gb, sem1).wait()
            for g in range(8):
                ridx = iota + g * 16
                w1 = [plsc.load_gather(wb1, [ridx, full16(j)])
                      for j in range(CH)]
                w2 = [plsc.load_gather(wb2, [ridx, full16(k)])
                      for k in range(CH)]
                for b in range(BH):
                    u = plsc.load_gather(gb, [ridx, full16(b * CH)]) * w1[0]
                    for j in range(1, CH):
                        u = u + plsc.load_gather(
                            gb, [ridx, full16(b * CH + j)]) * w1[j]
                    for k in range(CH):
                        plsc.store_scatter(cb, [ridx, full16(b * CH + k)],
                                           u * w2[k])
            pltpu.sync_copy(cb, acc.at[idxb.at[1]], add=True)
            return 0
        lax.fori_loop(0, C_FF // 16, body, 0)

    def fo_pass():
        def body(i, _):
            ck = i * 16 + s
            pltpu.sync_copy(idxFO_r.at[c, ck], idxb4)
            pltpu.sync_copy(wFO1_r.at[pl.ds(ck * CK, CK)], wb1)
            pltpu.sync_copy(wFO2_r.at[pl.ds(ck * CK, CK)], wb2)
            d1 = pltpu.async_copy(hid_r.at[idxb4.at[0]], gb, sem1)
            d2 = pltpu.async_copy(hid_r.at[idxb4.at[1]], gb2, sem2)
            d1.wait()
            d2.wait()

            def crow(r, _):
                cb[r, pl.ds(0, 16)] = zero16
                cb[r, pl.ds(16, 16)] = zero16
                return 0
            lax.fori_loop(0, CK, crow, 0)

            for g in range(8):
                ridx = iota + g * 16
                w1 = [plsc.load_gather(wb1, [ridx, full16(j)])
                      for j in range(CH)]
                w2 = [plsc.load_gather(wb2, [ridx, full16(j)])
                      for j in range(CH)]
                dst_v = idxb4[2, pl.ds(g * 16, 16)]
                rowv = lax.shift_right_logical(dst_v, 2)
                colb = lax.shift_left(lax.bitwise_and(dst_v, full16(3)), 3)
                idv[0, pl.ds(g * 16, 16)] = rowv
                for b in range(BH):
                    o = plsc.load_gather(gb, [ridx, full16(b * CH)]) * w1[0]
                    for j in range(1, CH):
                        o = o + plsc.load_gather(
                            gb, [ridx, full16(b * CH + j)]) * w1[j]
                    for j in range(CH):
                        o = o + plsc.load_gather(
                            gb2, [ridx, full16(b * CH + j)]) * w2[j]
                    plsc.store_scatter(cb, [ridx, colb + b], o)
            pltpu.sync_copy(cb, outacc.at[idv.at[0]], add=True)
            return 0
        lax.fori_loop(0, C_FO // 16, body, 0)

    if_pass(wIF1_r)
    plsc.subcore_barrier()
    elu_pass(0, zero_after=True)
    plsc.subcore_barrier()
    if_pass(wIF2_r)
    ff_pass()
    plsc.subcore_barrier()
    elu_pass(1, zero_after=False)
    plsc.subcore_barrier()
    fo_pass()
    plsc.subcore_barrier()
    pltpu.sync_copy(outacc.at[pl.ds(s * (OUTR // 16), OUTR // 16)],
                    out_r.at[c, pl.ds(s * (OUTR // 16), OUTR // 16)])


@jax.jit
def kernel(x, edge_index, w_in1, b1, w_out1, w_in2, b2, w_out2):
    src = edge_index[0].astype(jnp.int32)
    dst = edge_index[1].astype(jnp.int32)
    sFF, dFF = src[:E_FF], dst[:E_FF]
    sIF = src[E_FF:E_FF + E_IF] - N_FUNC
    dIF = dst[E_FF:E_FF + E_IF]
    sFO = src[E_FF + E_IF:]
    dFO = dst[E_FF + E_IF:] - (N_FUNC + N_IN)

    def padi(a, n):
        return jnp.concatenate(
            [a, jnp.zeros((n - a.shape[0],), jnp.int32)]).reshape(-1, CK)

    def padw(a, n):
        return jnp.concatenate(
            [a, jnp.zeros((n - a.shape[0], CH), jnp.float32)])

    sFFp, dFFp = padi(sFF, C_FF * CK), padi(dFF, C_FF * CK)
    sIFp, dIFp = padi(sIF, C_IF * CK), padi(dIF, C_IF * CK)
    sFOp, dFOp = padi(sFO, C_FO * CK), padi(dFO, C_FO * CK)

    idxFF = jnp.stack([jnp.stack([sFFp + cc * N_FUNC, dFFp], 1)
                       for cc in range(2)])
    idxIF = jnp.stack([jnp.stack([sIFp + cc * N_IN, dIFp], 1)
                       for cc in range(2)])
    zFO = jnp.zeros_like(sFOp)
    idxFO = jnp.stack(
        [jnp.stack([sFOp + cc * N_FUNC, sFOp + (2 + cc) * N_FUNC, dFOp, zFO], 1)
         for cc in range(2)])

    wFF1 = padw(w_out1[:E_FF], C_FF * CK)
    wFF2 = padw(w_in2[:E_FF], C_FF * CK)
    wIF1 = padw(w_in1[E_FF:E_FF + E_IF], C_IF * CK)
    wIF2 = padw(w_in2[E_FF:E_FF + E_IF], C_IF * CK)
    wFO1 = padw(w_out1[E_FF + E_IF:], C_FO * CK)
    wFO2 = padw(w_out2[E_FF + E_IF:], C_FO * CK)

    biasx = jnp.stack([
        jnp.tile(bb_.reshape(N_FUNC, 1, CH), (1, BH, 1)).reshape(N_FUNC, ROW)
        for bb_ in (b1, b2)])
    x2 = jnp.transpose(x.reshape(2, BH, N_IN), (0, 2, 1)).reshape(
        2 * N_IN, BH)

    mesh = plsc.VectorSubcoreMesh(core_axis_name="c", subcore_axis_name="s")
    out_type = (jax.ShapeDtypeStruct((2, OUTR, ROW), jnp.float32),
                jax.ShapeDtypeStruct((4 * N_FUNC, ROW), jnp.float32))
    scratch = [
        pltpu.VMEM((EC, ROW), jnp.float32),
        pltpu.VMEM((2, CK), jnp.int32),
        pltpu.VMEM((4, CK), jnp.int32),
        pltpu.VMEM((CK, CH), jnp.float32),
        pltpu.VMEM((CK, CH), jnp.float32),
        pltpu.VMEM((CK, ROW), jnp.float32),
        pltpu.VMEM((CK, ROW), jnp.float32),
        pltpu.VMEM((CK, ROW), jnp.float32),
        pltpu.VMEM((CK, BH), jnp.float32),
        pltpu.VMEM((EC, ROW), jnp.float32),
        pltpu.VMEM((EC, ROW), jnp.float32),
        pltpu.VMEM((1, CK), jnp.int32),
        pltpu.VMEM_SHARED((N_FUNC, ROW), jnp.float32),
        pltpu.VMEM_SHARED((OUTR, ROW), jnp.float32),
        pltpu.SemaphoreType.DMA,
        pltpu.SemaphoreType.DMA,
    ]
    out_hbm, _hid = pl.kernel(
        _body, out_type=out_type, mesh=mesh, scratch_types=scratch,
        compiler_params=pltpu.CompilerParams(
            needs_layout_passes=False, use_tc_tiling_on_sc=False),
        name="gsnn_sc")(
        x2, idxFF, wFF1, wFF2, idxIF, wIF1, wIF2,
        idxFO, wFO1, wFO2, biasx)

    op = out_hbm.reshape(2, OUTR * 4, BH)[:, :N_OUT, :]
    op = jnp.transpose(op, (0, 2, 1)).reshape(B, N_OUT)
    return jnp.concatenate(
        [jnp.zeros((B, N_FUNC + N_IN), jnp.float32), op], axis=1)
